# baseline (speedup 1.0000x reference)
import functools

import jax
import jax.numpy as jnp
from jax import lax
from jax.experimental import pallas as pl
from jax.experimental.pallas import tpu as pltpu
from jax.experimental.pallas import tpu_sc as plsc

DIM = 256
CDIM = 64
CSIZE = 8192
TOK_BLK = 512
C_CHUNK = 4096
N_TOK = 16 * 1024
N_BLK = N_TOK // TOK_BLK
IDX_CHUNK = 128


def _row_sumsq(v):
    a = v * v
    acc = a[:, 0:8]
    for i in range(1, 8):
        acc = acc + a[:, 8 * i : 8 * i + 8]
    acc = acc[:, 0:4] + acc[:, 4:8]
    acc = acc[:, 0:2] + acc[:, 2:4]
    return acc[:, 0:1] + acc[:, 1:2]


def _dist_argmax_body(x_ref, w_in_ref, b_in_ref, embed_ref, ind_ref, z_ref):
    z = (
        jnp.dot(x_ref[...], w_in_ref[...], preferred_element_type=jnp.float32)
        + b_in_ref[0, :][None, :]
    )
    z_ref[...] = z
    zz = _row_sumsq(z)
    zb = z.astype(jnp.bfloat16)

    best = jnp.full((TOK_BLK,), -jnp.inf, dtype=jnp.float32)
    besti = jnp.zeros((TOK_BLK,), dtype=jnp.int32)
    for c in range(CSIZE // C_CHUNK):
        e = embed_ref[c * C_CHUNK : (c + 1) * C_CHUNK, :]
        mm = lax.dot_general(
            zb,
            e.astype(jnp.bfloat16),
            (((1,), (1,)), ((), ())),
            preferred_element_type=jnp.float32,
        )
        ee = _row_sumsq(e)[:, 0]
        dist = -(zz - 2.0 * mm + ee[None, :])
        cmax = jnp.max(dist, axis=1)
        iot = lax.broadcasted_iota(jnp.int32, (TOK_BLK, C_CHUNK), 1)
        cand = jnp.where(dist == cmax[:, None], iot, C_CHUNK)
        cidx = jnp.min(cand, axis=1) + c * C_CHUNK
        upd = cmax > best
        best = jnp.where(upd, cmax, best).astype(jnp.bfloat16).astype(jnp.float32)
        besti = jnp.where(upd, cidx, besti)

    ind_ref[0, 0, :] = besti


def _proj_out_body(q_ref, z_ref, w_out_ref, b_out_ref, out_ref, lsum_ref):
    q = q_ref[:, 0:CDIM]
    z = z_ref[...]
    r = q - z
    qst = z + r
    out_ref[...] = (
        jnp.dot(qst, w_out_ref[...], preferred_element_type=jnp.float32)
        + b_out_ref[0, :][None, :]
    )
    lsum_ref[0, 0, :] = jnp.broadcast_to(jnp.sum(r * r), (128,))


def _sc_gather(table, ind2d):
    info = plsc.get_sparse_core_info()
    nw = info.num_cores * info.num_subcores
    nc = info.num_cores
    b_per_w = N_TOK // nw
    n_chunk = b_per_w // IDX_CHUNK

    @functools.partial(
        pl.kernel,
        mesh=plsc.VectorSubcoreMesh(core_axis_name="c", subcore_axis_name="s"),
        out_type=jax.ShapeDtypeStruct((N_TOK, 128), jnp.float32),
        scratch_types=[
            pltpu.VMEM((n_chunk, IDX_CHUNK), jnp.int32),
            pltpu.VMEM((IDX_CHUNK, 128), jnp.float32),
            pltpu.SemaphoreType.DMA,
        ],
    )
    def gather_k(ind_hbm, table_hbm, out_hbm, idx_v, rows_v, sem):
        wid = lax.axis_index("s") * nc + lax.axis_index("c")
        base = wid * b_per_w
        pltpu.sync_copy(ind_hbm.at[pl.ds(wid * n_chunk, n_chunk)], idx_v)
        for j in range(n_chunk):
            pltpu.async_copy(table_hbm.at[idx_v.at[j]], rows_v, sem).wait()
            pltpu.sync_copy(rows_v, out_hbm.at[pl.ds(base + j * IDX_CHUNK, IDX_CHUNK)])

    return gather_k(ind2d, table)


@jax.jit
def kernel(x, W_in, b_in, W_out, b_out, embed):
    xf = x.reshape(N_TOK, DIM)

    ind_blocks, zf = pl.pallas_call(
        _dist_argmax_body,
        grid=(N_BLK,),
        in_specs=[
            pl.BlockSpec((TOK_BLK, DIM), lambda i: (i, 0)),
            pl.BlockSpec((DIM, CDIM), lambda i: (0, 0)),
            pl.BlockSpec((1, CDIM), lambda i: (0, 0)),
            pl.BlockSpec((CSIZE, CDIM), lambda i: (0, 0)),
        ],
        out_specs=[
            pl.BlockSpec((1, 1, TOK_BLK), lambda i: (i, 0, 0)),
            pl.BlockSpec((TOK_BLK, CDIM), lambda i: (i, 0)),
        ],
        out_shape=[
            jax.ShapeDtypeStruct((N_BLK, 1, TOK_BLK), jnp.int32),
            jax.ShapeDtypeStruct((N_TOK, CDIM), jnp.float32),
        ],
    )(xf, W_in, b_in.reshape(1, CDIM), embed)

    embed_pad = jnp.pad(embed, ((0, 0), (0, 128 - CDIM)))
    ind2d = ind_blocks.reshape(N_TOK // IDX_CHUNK, IDX_CHUNK)
    quant_pad = _sc_gather(embed_pad, ind2d)

    out_flat, lsum = pl.pallas_call(
        _proj_out_body,
        grid=(N_BLK,),
        in_specs=[
            pl.BlockSpec((TOK_BLK, 128), lambda i: (i, 0)),
            pl.BlockSpec((TOK_BLK, CDIM), lambda i: (i, 0)),
            pl.BlockSpec((CDIM, DIM), lambda i: (0, 0)),
            pl.BlockSpec((1, DIM), lambda i: (0, 0)),
        ],
        out_specs=[
            pl.BlockSpec((TOK_BLK, DIM), lambda i: (i, 0)),
            pl.BlockSpec((1, 1, 128), lambda i: (i, 0, 0)),
        ],
        out_shape=[
            jax.ShapeDtypeStruct((N_TOK, DIM), jnp.float32),
            jax.ShapeDtypeStruct((N_BLK, 1, 128), jnp.float32),
        ],
    )(quant_pad, zf, W_out, b_out.reshape(1, DIM))

    out = out_flat.reshape(x.shape)
    embed_ind = ind_blocks.reshape(x.shape[:-1])
    loss = jnp.sum(lsum[:, 0, 0]) * (1.0 / (N_TOK * CDIM))
    return out, embed_ind, loss
